# baseline (speedup 1.0000x reference)
import jax
import jax.numpy as jnp
from jax import lax
from jax.experimental import pallas as pl
from jax.experimental.pallas import tpu as pltpu
from jax.experimental.pallas import tpu_sc as plsc

K = 5
D = 128
HD = 64
N = 10000
E = 320000
NC = 2
NS = 16
L = 16
NT = NC * NS
N_PAD = 10240
RPT = N_PAD // NS
EPT = E // NT
CH = 128
NCHS = 160
EPS = NCHS * CH
E_PAD = EPS * NS
BN = 1024


def _mesh():
    return plsc.VectorSubcoreMesh(
        core_axis_name="c", subcore_axis_name="s", num_cores=NC, num_subcores=NS
    )


def _norm_body(src_hbm, dst_hbm, ew_hbm, norm_hbm, ai, bi, af, cf, deg, dinv):
    c = lax.axis_index("c")
    s = lax.axis_index("s")
    wid = s * NC + c

    @pl.loop(0, N // L)
    def _zero(i):
        deg[pl.ds(i * L, L)] = jnp.zeros((L,), jnp.float32)

    @pl.loop(0, NT)
    def _slab(sl):
        pltpu.sync_copy(dst_hbm.at[pl.ds(sl * EPT, EPT)], ai)
        pltpu.sync_copy(ew_hbm.at[pl.ds(sl * EPT, EPT)], af)

        @pl.loop(0, EPT // L, unroll=4)
        def _acc(i)    :
            slc = pl.ds(i * L, L)
            plsc.addupdate_scatter(deg, [ai[slc]], af[slc])

    @pl.loop(0, N // L)
    def _dinv(i):
        slc = pl.ds(i * L, L)
        d = deg[slc]
        ds_ = jnp.where(d > 0.0, d, 1.0)
        yi = 0x5F3759DF - lax.shift_right_logical(plsc.bitcast(ds_, jnp.int32), 1)
        y = plsc.bitcast(yi, jnp.float32)
        y = y * (1.5 - 0.5 * ds_ * y * y)
        y = y * (1.5 - 0.5 * ds_ * y * y)
        y = y * (1.5 - 0.5 * ds_ * y * y)
        dinv[slc] = jnp.where(d > 0.0, y, 0.0)

    base = wid * EPT
    pltpu.sync_copy(src_hbm.at[pl.ds(base, EPT)], ai)
    pltpu.sync_copy(dst_hbm.at[pl.ds(base, EPT)], bi)
    pltpu.sync_copy(ew_hbm.at[pl.ds(base, EPT)], af)

    @pl.loop(0, EPT // L, unroll=4)
    def _nrm(i):
        slc = pl.ds(i * L, L)
        nv = plsc.load_gather(dinv, [ai[slc]]) * af[slc] * plsc.load_gather(dinv, [bi[slc]])
        cf[slc] = nv

    pltpu.sync_copy(cf, norm_hbm.at[pl.ds(base, EPT)])


def _sc_norm(src, dst, ew):
    return pl.kernel(
        _norm_body,
        out_type=jax.ShapeDtypeStruct((E,), jnp.float32),
        mesh=_mesh(),
        compiler_params=pltpu.CompilerParams(needs_layout_passes=False),
        scratch_types=[
            pltpu.VMEM((EPT,), jnp.int32),
            pltpu.VMEM((EPT,), jnp.int32),
            pltpu.VMEM((EPT,), jnp.float32),
            pltpu.VMEM((EPT,), jnp.float32),
            pltpu.VMEM((N,), jnp.float32),
            pltpu.VMEM((N,), jnp.float32),
        ],
    )(src, dst, ew)


def _layer_body(hcat_hbm, src_hbm, dst_hbm, nrm_hbm, out_hbm,
                src_v, dst_v, nrm_v, rows0, rows1, rows2, rows3, zb,
                acc, gs0, gs1, gs2, gs3, ss0, ss1, ss2, ss3):
    c = lax.axis_index("c")
    s = lax.axis_index("s")
    row0 = s * RPT

    @pl.loop(0, CH * (HD // L))
    def _z(i):
        r = i // (HD // L)
        k = i % (HD // L)
        zb[r, pl.ds(k * L, L)] = jnp.zeros((L,), jnp.float32)

    pltpu.sync_copy(src_hbm.at[c * NS + s], src_v)
    for r in (2, 3):
        for cb in range(CH // L):
            dst_v[r, pl.ds(cb * L, L)] = jnp.zeros((L,), jnp.int32)
    for q in range(RPT // CH):
        pltpu.sync_copy(zb, acc.at[pl.ds(row0 + q * CH, CH)])
    plsc.subcore_barrier()

    bufs = ((rows0, gs0, ss0), (rows1, gs1, ss1),
            (rows2, gs2, ss2), (rows3, gs3, ss3))
    NB = 4

    for k in range(K):
        table = hcat_hbm if k == 0 else out_hbm
        if k >= 2:
            @pl.loop(0, NCHS)
            def _bump(j):
                for cb in range(CH // L):
                    slc = pl.ds(cb * L, L)
                    src_v[j, slc] = src_v[j, slc] + jnp.full((L,), 2 * N_PAD, jnp.int32)

        ebase = s * EPS

        def _prefetch(chunk, buf, gsem):
            pltpu.async_copy(table.at[src_v.at[chunk]], bufs[buf][0], gsem)
            pltpu.async_copy(dst_hbm.at[pl.ds(ebase + chunk * CH, CH)],
                             dst_v.at[buf], gsem)
            pltpu.async_copy(nrm_hbm.at[pl.ds(ebase + chunk * CH, CH)],
                             nrm_v.at[pl.ds(buf * CH, CH)], gsem)

        def _gwait(chunk, buf, gsem):
            pltpu.make_async_copy(table.at[src_v.at[chunk]], bufs[buf][0], gsem).wait()
            pltpu.make_async_copy(dst_hbm.at[pl.ds(ebase, CH)],
                                  dst_v.at[buf], gsem).wait()
            pltpu.make_async_copy(nrm_hbm.at[pl.ds(ebase, CH)],
                                  nrm_v.at[pl.ds(buf * CH, CH)], gsem).wait()

        pltpu.async_copy(zb, acc.at[dst_v.at[2]], ss2, add=True)
        pltpu.async_copy(zb, acc.at[dst_v.at[3]], ss3, add=True)
        _prefetch(jnp.int32(0), 0, gs0)
        _prefetch(jnp.int32(1), 1, gs1)

        @pl.loop(0, NCHS // NB)
        def _main(g):
            for b in range(NB):
                ch = g * NB + b
                p = (b + 2) % NB
                rows, gsem, ssem = bufs[b]
                prows, pgsem, pssem = bufs[p]
                pltpu.make_async_copy(prows, acc.at[dst_v.at[p]], pssem).wait()
                nxt = ch + 2
                nxt = jnp.where(nxt >= NCHS, nxt - NCHS, nxt)
                _prefetch(nxt, p, pgsem)
                _gwait(ch, b, gsem)

                base = b * CH

                @pl.loop(0, CH, unroll=4)
                def _scale(j):
                    nv = plsc.load_gather(nrm_v, [jnp.full((L,), base + j, jnp.int32)])
                    for cb in range(HD // L):
                        slc = pl.ds(cb * L, L)
                        rows[j, slc] = rows[j, slc] * nv

                pltpu.async_copy(rows, acc.at[dst_v.at[b]], ssem, add=True)

        _gwait(jnp.int32(0), 0, gs0)
        _gwait(jnp.int32(1), 1, gs1)
        pltpu.make_async_copy(rows2, acc.at[dst_v.at[2]], ss2).wait()
        pltpu.make_async_copy(rows3, acc.at[dst_v.at[3]], ss3).wait()
        plsc.subcore_barrier()

        pltpu.sync_copy(acc.at[pl.ds(row0, RPT)],
                        out_hbm.at[pl.ds(k * 2 * N_PAD + c * N_PAD + row0, RPT)])
        if k < K - 1:
            for q in range(RPT // CH):
                pltpu.sync_copy(zb, acc.at[pl.ds(row0 + q * CH, CH)])
        plsc.subcore_barrier()


def _sc_layer(hcat, srco3, dst3, nrmf):
    out = pl.kernel(
        _layer_body,
        out_type=jax.ShapeDtypeStruct((K * 2 * N_PAD, HD), jnp.float32),
        mesh=_mesh(),
        compiler_params=pltpu.CompilerParams(
            needs_layout_passes=False, use_tc_tiling_on_sc=False
        ),
        scratch_types=[
            pltpu.VMEM((NCHS, CH), jnp.int32),
            pltpu.VMEM((4, CH), jnp.int32),
            pltpu.VMEM((4 * CH,), jnp.float32),
            pltpu.VMEM((CH, HD), jnp.float32),
            pltpu.VMEM((CH, HD), jnp.float32),
            pltpu.VMEM((CH, HD), jnp.float32),
            pltpu.VMEM((CH, HD), jnp.float32),
            pltpu.VMEM((CH, HD), jnp.float32),
            pltpu.VMEM_SHARED((N_PAD, HD), jnp.float32),
            pltpu.SemaphoreType.DMA,
            pltpu.SemaphoreType.DMA,
            pltpu.SemaphoreType.DMA,
            pltpu.SemaphoreType.DMA,
            pltpu.SemaphoreType.DMA,
            pltpu.SemaphoreType.DMA,
            pltpu.SemaphoreType.DMA,
            pltpu.SemaphoreType.DMA,
        ],
    )(hcat, srco3, dst3, nrmf)
    return out.reshape(K, 2 * N_PAD, HD)


def _combine_body(*refs):
    hs, wc_ref, b_ref, ost, ofl = refs[: 2 * (K + 1)], refs[-4], refs[-3], refs[-2], refs[-1]
    hblk = jnp.concatenate([h[...].reshape(BN, HD) for h in hs], axis=1)
    acc = jnp.dot(hblk, wc_ref[...], preferred_element_type=jnp.float32)
    acc = acc + b_ref[...]
    acc = jnp.where(acc >= 0, acc, 0.01 * acc)
    ofl[...] = acc
    ost[0] = acc[:, :HD]
    ost[1] = acc[:, HD:]


def _combine(hcat, houts, W, b):
    wc = W.reshape((K + 1) * D, D)
    in_specs = [
        pl.BlockSpec((1, BN, HD), lambda i: (0, i, 0)),
        pl.BlockSpec((1, BN, HD), lambda i: (0, N_PAD // BN + i, 0)),
    ]
    for k in range(K):
        in_specs.append(pl.BlockSpec((1, BN, HD), lambda i, k=k: (k, i, 0)))
        in_specs.append(
            pl.BlockSpec((1, BN, HD), lambda i, k=k: (k, N_PAD // BN + i, 0))
        )
    in_specs.append(pl.BlockSpec(((K + 1) * D, D), lambda i: (0, 0)))
    in_specs.append(pl.BlockSpec((1, D), lambda i: (0, 0)))
    hcat3 = hcat.reshape(1, 2 * N_PAD, HD)
    out_st, out_fl = pl.pallas_call(
        _combine_body,
        grid=(N_PAD // BN,),
        in_specs=in_specs,
        out_specs=[
            pl.BlockSpec((2, BN, HD), lambda i: (0, i, 0)),
            pl.BlockSpec((BN, D), lambda i: (i, 0)),
        ],
        out_shape=[
            jax.ShapeDtypeStruct((2, N_PAD, HD), jnp.float32),
            jax.ShapeDtypeStruct((N_PAD, D), jnp.float32),
        ],
    )(hcat3, hcat3, *[houts for _ in range(2 * K)], wc, b.reshape(1, D))
    return out_st.reshape(2 * N_PAD, HD), out_fl


def kernel(x, edge_index, edge_weight, W1, b1, W2, b2):
    src = edge_index[0]
    dst = edge_index[1]
    norm = _sc_norm(src, dst, edge_weight)

    pad = E_PAD - E
    srcp = jnp.pad(src, (0, pad))
    core_off = (jnp.arange(NC, dtype=jnp.int32) * N_PAD)[:, None]
    srco3 = (srcp[None, :] + core_off).reshape(NC * NS, NCHS, CH)
    dstf = jnp.pad(dst, (0, pad))
    nrmf = jnp.pad(norm, (0, pad))

    xp = jnp.pad(x, ((0, N_PAD - N), (0, 0)))
    hcat = jnp.concatenate([xp[:, :HD], xp[:, HD:]], axis=0)

    flat = None
    for (W, b) in ((W1, b1), (W2, b2)):
        houts = _sc_layer(hcat, srco3, dstf, nrmf)
        hcat, flat = _combine(hcat, houts, W, b)
    return flat[:N]


# speedup vs baseline: 1.0605x; 1.0605x over previous
import jax
import jax.numpy as jnp
from jax import lax
from jax.experimental import pallas as pl
from jax.experimental.pallas import tpu as pltpu
from jax.experimental.pallas import tpu_sc as plsc

K = 5
D = 128
HD = 64
N = 10000
E = 320000
NC = 2
NS = 16
L = 16
NT = NC * NS
N_PAD = 10240
RPT = N_PAD // NS
EPT = E // NT
CH = 128
NCHS = 160
BCH = 16
NBLK = NCHS // BCH
EPS = NCHS * CH
E_PAD = EPS * NS
BN = 1024


def _mesh():
    return plsc.VectorSubcoreMesh(
        core_axis_name="c", subcore_axis_name="s", num_cores=NC, num_subcores=NS
    )


def _norm_body(src_hbm, dst_hbm, ew_hbm, norm_hbm, ai, bi, af, cf, deg, dinv):
    c = lax.axis_index("c")
    s = lax.axis_index("s")
    wid = s * NC + c

    @pl.loop(0, N // L)
    def _zero(i):
        deg[pl.ds(i * L, L)] = jnp.zeros((L,), jnp.float32)

    @pl.loop(0, NT)
    def _slab(sl):
        pltpu.sync_copy(dst_hbm.at[pl.ds(sl * EPT, EPT)], ai)
        pltpu.sync_copy(ew_hbm.at[pl.ds(sl * EPT, EPT)], af)

        @pl.loop(0, EPT // L, unroll=4)
        def _acc(i):
            slc = pl.ds(i * L, L)
            plsc.addupdate_scatter(deg, [ai[slc]], af[slc])

    @pl.loop(0, N // L)
    def _dinv(i):
        slc = pl.ds(i * L, L)
        d = deg[slc]
        ds_ = jnp.where(d > 0.0, d, 1.0)
        yi = 0x5F3759DF - lax.shift_right_logical(plsc.bitcast(ds_, jnp.int32), 1)
        y = plsc.bitcast(yi, jnp.float32)
        y = y * (1.5 - 0.5 * ds_ * y * y)
        y = y * (1.5 - 0.5 * ds_ * y * y)
        y = y * (1.5 - 0.5 * ds_ * y * y)
        dinv[slc] = jnp.where(d > 0.0, y, 0.0)

    base = wid * EPT
    pltpu.sync_copy(src_hbm.at[pl.ds(base, EPT)], ai)
    pltpu.sync_copy(dst_hbm.at[pl.ds(base, EPT)], bi)
    pltpu.sync_copy(ew_hbm.at[pl.ds(base, EPT)], af)

    @pl.loop(0, EPT // L, unroll=4)
    def _nrm(i):
        slc = pl.ds(i * L, L)
        nv = plsc.load_gather(dinv, [ai[slc]]) * af[slc] * plsc.load_gather(dinv, [bi[slc]])
        cf[slc] = nv

    pltpu.sync_copy(cf, norm_hbm.at[pl.ds(base, EPT)])


def _sc_norm(src, dst, ew):
    return pl.kernel(
        _norm_body,
        out_type=jax.ShapeDtypeStruct((E,), jnp.float32),
        mesh=_mesh(),
        compiler_params=pltpu.CompilerParams(needs_layout_passes=False),
        scratch_types=[
            pltpu.VMEM((EPT,), jnp.int32),
            pltpu.VMEM((EPT,), jnp.int32),
            pltpu.VMEM((EPT,), jnp.float32),
            pltpu.VMEM((EPT,), jnp.float32),
            pltpu.VMEM((N,), jnp.float32),
            pltpu.VMEM((N,), jnp.float32),
        ],
    )(src, dst, ew)


def _layer_body(hcat_hbm, src_hbm, dst_hbm, nrm_hbm, out_hbm,
                src_v, dst_v, nrm_v, rows0, rows1, rows2, rows3, zb,
                acc, gs0, gs1, gs2, gs3, ss0, ss1, ss2, ss3, bs0, bs1):
    c = lax.axis_index("c")
    s = lax.axis_index("s")
    row0 = s * RPT

    @pl.loop(0, CH * (HD // L))
    def _z(i):
        r = i // (HD // L)
        k = i % (HD // L)
        zb[r, pl.ds(k * L, L)] = jnp.zeros((L,), jnp.float32)

    pltpu.sync_copy(src_hbm.at[s], src_v)
    coff = c * N_PAD

    @pl.loop(0, NCHS)
    def _off(j):
        for cb in range(CH // L):
            slc = pl.ds(cb * L, L)
            src_v[j, slc] = src_v[j, slc] + jnp.full((L,), coff, jnp.int32)

    for r in (BCH + 2, BCH + 3):
        for cb in range(CH // L):
            dst_v[r, pl.ds(cb * L, L)] = jnp.zeros((L,), jnp.int32)
    for q in range(RPT // CH):
        pltpu.sync_copy(zb, acc.at[pl.ds(row0 + q * CH, CH)])
    plsc.subcore_barrier()

    bufs = ((rows0, gs0, ss0), (rows1, gs1, ss1),
            (rows2, gs2, ss2), (rows3, gs3, ss3))
    NB = 4

    for q in range(RPT // CH):
        pltpu.sync_copy(hcat_hbm.at[pl.ds(c * N_PAD + row0 + q * CH, CH)], rows0)
        pltpu.sync_copy(rows0, out_hbm.at[pl.ds(c * N_PAD + row0 + q * CH, CH)])
    plsc.subcore_barrier()

    @pl.loop(0, K)
    def _hop(k):
        table = out_hbm

        def _blk_issue(blk, sem):
            par = (blk % 2) * BCH
            pltpu.async_copy(dst_hbm.at[s * NBLK + blk],
                             dst_v.at[pl.ds(par, BCH)], sem)
            pltpu.async_copy(nrm_hbm.at[s * NBLK + blk],
                             nrm_v.at[pl.ds(par, BCH)], sem)

        def _blk_wait(blk, sem):
            pltpu.make_async_copy(dst_hbm.at[s * NBLK],
                                  dst_v.at[pl.ds(0, BCH)], sem).wait()
            pltpu.make_async_copy(nrm_hbm.at[s * NBLK],
                                  nrm_v.at[pl.ds(0, BCH)], sem).wait()

        _blk_issue(jnp.int32(0), bs0)
        pltpu.async_copy(zb, acc.at[dst_v.at[BCH + 2]], ss2, add=True)
        pltpu.async_copy(zb, acc.at[dst_v.at[BCH + 3]], ss3, add=True)
        pltpu.async_copy(table.at[src_v.at[0]], rows0, gs0)
        pltpu.async_copy(table.at[src_v.at[1]], rows1, gs1)

        @pl.loop(0, NCHS // NB)
        def _main(g):
            for b in range(NB):
                ch = g * NB + b
                p = (b + 2) % NB
                rows, gsem, ssem = bufs[b]
                prows, pgsem, pssem = bufs[p]

                if b == 0:
                    bnd = g % (BCH // NB) == 0
                    bpar = (g // (BCH // NB)) % 2
                    blk = ch // BCH

                    @pl.when(bnd & (bpar == 0))
                    def _blkw0():
                        _blk_wait(blk, bs0)

                    @pl.when(bnd & (bpar == 1))
                    def _blkw1():
                        _blk_wait(blk, bs1)

                drow = ((ch // BCH) % 2) * BCH + (ch % BCH)

                pltpu.make_async_copy(prows, acc.at[dst_v.at[drow]], pssem).wait()

                if b == 1:
                    bnd = g % (BCH // NB) == 0
                    bpar = (g // (BCH // NB)) % 2
                    nblk = ch // BCH + 1
                    nblk = jnp.where(nblk >= NBLK, 0, nblk)

                    @pl.when(bnd & (bpar == 0))
                    def _blki0():
                        _blk_issue(nblk, bs1)

                    @pl.when(bnd & (bpar == 1))
                    def _blki1():
                        _blk_issue(nblk, bs0)
                nxt = ch + 2
                nxt = jnp.where(nxt >= NCHS, nxt - NCHS, nxt)
                pltpu.async_copy(table.at[src_v.at[nxt]], prows, pgsem)
                pltpu.make_async_copy(table.at[src_v.at[ch]], rows, gsem).wait()

                @pl.loop(0, CH // L)
                def _grp(jg):
                    nv16 = nrm_v[drow, pl.ds(jg * L, L)]
                    r0 = jg * L
                    for q in range(4):
                        nvs = [nv16[jnp.full((L,), 4 * q + t, jnp.int32)]
                               for t in range(4)]
                        for cb in range(HD // L):
                            for t in range(4):
                                slc = pl.ds(cb * L, L)
                                r = r0 + 4 * q + t
                                rows[r, slc] = rows[r, slc] * nvs[t]

                pltpu.async_copy(rows, acc.at[dst_v.at[drow]], ssem, add=True)

        _blk_wait(jnp.int32(0), bs0)
        pltpu.make_async_copy(table.at[src_v.at[0]], rows0, gs0).wait()
        pltpu.make_async_copy(table.at[src_v.at[1]], rows1, gs1).wait()
        pltpu.make_async_copy(rows2, acc.at[dst_v.at[0]], ss2).wait()
        pltpu.make_async_copy(rows3, acc.at[dst_v.at[0]], ss3).wait()
        plsc.subcore_barrier()

        pltpu.sync_copy(acc.at[pl.ds(row0, RPT)],
                        out_hbm.at[pl.ds((k + 1) * (2 * N_PAD) + c * N_PAD + row0, RPT)])
        for q in range(RPT // CH):
            pltpu.sync_copy(zb, acc.at[pl.ds(row0 + q * CH, CH)])

        @pl.loop(0, NCHS)
        def _bump(j):
            for cb in range(CH // L):
                slc = pl.ds(cb * L, L)
                src_v[j, slc] = src_v[j, slc] + jnp.full((L,), 2 * N_PAD, jnp.int32)

        plsc.subcore_barrier()


def _sc_layer(hcat, srco3, dst3, nrmf):
    out = pl.kernel(
        _layer_body,
        out_type=jax.ShapeDtypeStruct(((K + 1) * 2 * N_PAD, HD), jnp.float32),
        mesh=_mesh(),
        compiler_params=pltpu.CompilerParams(
            needs_layout_passes=False, use_tc_tiling_on_sc=False
        ),
        scratch_types=[
            pltpu.VMEM((NCHS, CH), jnp.int32),
            pltpu.VMEM((2 * BCH, CH), jnp.int32),
            pltpu.VMEM((2 * BCH, CH), jnp.float32),
            pltpu.VMEM((CH, HD), jnp.float32),
            pltpu.VMEM((CH, HD), jnp.float32),
            pltpu.VMEM((CH, HD), jnp.float32),
            pltpu.VMEM((CH, HD), jnp.float32),
            pltpu.VMEM((CH, HD), jnp.float32),
            pltpu.VMEM_SHARED((N_PAD, HD), jnp.float32),
            pltpu.SemaphoreType.DMA,
            pltpu.SemaphoreType.DMA,
            pltpu.SemaphoreType.DMA,
            pltpu.SemaphoreType.DMA,
            pltpu.SemaphoreType.DMA,
            pltpu.SemaphoreType.DMA,
            pltpu.SemaphoreType.DMA,
            pltpu.SemaphoreType.DMA,
            pltpu.SemaphoreType.DMA,
            pltpu.SemaphoreType.DMA,
        ],
    )(hcat, srco3, dst3, nrmf)
    return out.reshape(K + 1, 2 * N_PAD, HD)


def _combine_body(*refs):
    hs, wc_ref, b_ref, ost, ofl = refs[: 2 * (K + 1)], refs[-4], refs[-3], refs[-2], refs[-1]
    hblk = jnp.concatenate([h[...].reshape(BN, HD) for h in hs], axis=1)
    acc = jnp.dot(hblk, wc_ref[...], preferred_element_type=jnp.float32)
    acc = acc + b_ref[...]
    acc = jnp.where(acc >= 0, acc, 0.01 * acc)
    ofl[...] = acc
    ost[0] = acc[:, :HD]
    ost[1] = acc[:, HD:]


def _combine(houts, W, b):
    wc = W.reshape((K + 1) * D, D)
    in_specs = []
    for k in range(K + 1):
        in_specs.append(pl.BlockSpec((1, BN, HD), lambda i, k=k: (k, i, 0)))
        in_specs.append(
            pl.BlockSpec((1, BN, HD), lambda i, k=k: (k, N_PAD // BN + i, 0))
        )
    in_specs.append(pl.BlockSpec(((K + 1) * D, D), lambda i: (0, 0)))
    in_specs.append(pl.BlockSpec((1, D), lambda i: (0, 0)))
    out_st, out_fl = pl.pallas_call(
        _combine_body,
        grid=(N_PAD // BN,),
        in_specs=in_specs,
        out_specs=[
            pl.BlockSpec((2, BN, HD), lambda i: (0, i, 0)),
            pl.BlockSpec((BN, D), lambda i: (i, 0)),
        ],
        out_shape=[
            jax.ShapeDtypeStruct((2, N_PAD, HD), jnp.float32),
            jax.ShapeDtypeStruct((N_PAD, D), jnp.float32),
        ],
    )(*[houts for _ in range(2 * (K + 1))], wc, b.reshape(1, D))
    return out_st.reshape(2 * N_PAD, HD), out_fl


def kernel(x, edge_index, edge_weight, W1, b1, W2, b2):
    src = edge_index[0]
    dst = edge_index[1]
    norm = _sc_norm(src, dst, edge_weight)

    pad = E_PAD - E
    src3 = jnp.pad(src, (0, pad)).reshape(NS, NCHS, CH)
    dst3 = jnp.pad(dst, (0, pad)).reshape(NS * NBLK, BCH, CH)
    nrm3 = jnp.pad(norm, (0, pad)).reshape(NS * NBLK, BCH, CH)

    xp = jnp.pad(x, ((0, N_PAD - N), (0, 0)))
    hcat = jnp.concatenate([xp[:, :HD], xp[:, HD:]], axis=0)

    flat = None
    for (W, b) in ((W1, b1), (W2, b2)):
        houts = _sc_layer(hcat, src3, dst3, nrm3)
        hcat, flat = _combine(houts, W, b)
    return flat[:N]


# speedup vs baseline: 1.3705x; 1.2923x over previous
import jax
import jax.numpy as jnp
from jax import lax
from jax.experimental import pallas as pl
from jax.experimental.pallas import tpu as pltpu
from jax.experimental.pallas import tpu_sc as plsc

K = 5
D = 128
HD = 64
N = 10000
E = 320000
NC = 2
NS = 16
L = 16
NT = NC * NS
N_PAD = 10240
RPT = N_PAD // NS
EPT = E // NT
CH = 128
NCHS = 158
EPS = NCHS * CH
E_PAD = EPS * NS
BN = 1024


def _mesh():
    return plsc.VectorSubcoreMesh(
        core_axis_name="c", subcore_axis_name="s", num_cores=NC, num_subcores=NS
    )


def _norm_body(src_hbm, dst_hbm, ew_hbm, norm_hbm, ai, bi, af, cf, deg, dinv):
    c = lax.axis_index("c")
    s = lax.axis_index("s")
    wid = s * NC + c

    @pl.loop(0, N // L)
    def _zero(i):
        deg[pl.ds(i * L, L)] = jnp.zeros((L,), jnp.float32)

    @pl.loop(0, NT)
    def _slab(sl):
        pltpu.sync_copy(dst_hbm.at[pl.ds(sl * EPT, EPT)], ai)
        pltpu.sync_copy(ew_hbm.at[pl.ds(sl * EPT, EPT)], af)

        @pl.loop(0, EPT // L, unroll=4)
        def _acc(i):
            slc = pl.ds(i * L, L)
            plsc.addupdate_scatter(deg, [ai[slc]], af[slc])

    @pl.loop(0, N // L)
    def _dinv(i):
        slc = pl.ds(i * L, L)
        d = deg[slc]
        ds_ = jnp.where(d > 0.0, d, 1.0)
        yi = 0x5F3759DF - lax.shift_right_logical(plsc.bitcast(ds_, jnp.int32), 1)
        y = plsc.bitcast(yi, jnp.float32)
        y = y * (1.5 - 0.5 * ds_ * y * y)
        y = y * (1.5 - 0.5 * ds_ * y * y)
        y = y * (1.5 - 0.5 * ds_ * y * y)
        dinv[slc] = jnp.where(d > 0.0, y, 0.0)

    base = wid * EPT
    pltpu.sync_copy(src_hbm.at[pl.ds(base, EPT)], ai)
    pltpu.sync_copy(dst_hbm.at[pl.ds(base, EPT)], bi)
    pltpu.sync_copy(ew_hbm.at[pl.ds(base, EPT)], af)

    @pl.loop(0, EPT // L, unroll=4)
    def _nrm(i):
        slc = pl.ds(i * L, L)
        nv = plsc.load_gather(dinv, [ai[slc]]) * af[slc] * plsc.load_gather(dinv, [bi[slc]])
        cf[slc] = nv

    pltpu.sync_copy(cf, norm_hbm.at[pl.ds(base, EPT)])


def _sc_norm(src, dst, ew):
    return pl.kernel(
        _norm_body,
        out_type=jax.ShapeDtypeStruct((E,), jnp.float32),
        mesh=_mesh(),
        compiler_params=pltpu.CompilerParams(needs_layout_passes=False),
        scratch_types=[
            pltpu.VMEM((EPT,), jnp.int32),
            pltpu.VMEM((EPT,), jnp.int32),
            pltpu.VMEM((EPT,), jnp.float32),
            pltpu.VMEM((EPT,), jnp.float32),
            pltpu.VMEM((N,), jnp.float32),
            pltpu.VMEM((N,), jnp.float32),
        ],
    )(src, dst, ew)


def _layer_body(hcat_hbm, src_hbm, dst_hbm, nrm_hbm, out_hbm,
                src_v, dst_v, nrm_v, rows0, rows1, zb,
                acc, gs0, gs1, ss0, ss1):
    c = lax.axis_index("c")
    s = lax.axis_index("s")
    row0 = s * RPT

    @pl.loop(0, CH * (HD // L))
    def _z(i):
        r = i // (HD // L)
        k = i % (HD // L)
        zb[r, pl.ds(k * L, L)] = jnp.zeros((L,), jnp.float32)

    pltpu.sync_copy(src_hbm.at[c * NS + s], src_v)
    pltpu.sync_copy(dst_hbm.at[s], dst_v)
    pltpu.sync_copy(nrm_hbm.at[s], nrm_v)
    for q in range(RPT // CH):
        pltpu.sync_copy(zb, acc.at[pl.ds(row0 + q * CH, CH)])
    plsc.subcore_barrier()

    bufs = ((rows0, gs0, ss0), (rows1, gs1, ss1))

    for k in range(K):
        table = hcat_hbm if k == 0 else out_hbm
        if k >= 2:
            @pl.loop(0, NCHS)
            def _bump(j):
                for cb in range(CH // L):
                    slc = pl.ds(cb * L, L)
                    src_v[j, slc] = src_v[j, slc] + jnp.full((L,), 2 * N_PAD, jnp.int32)

        pltpu.async_copy(zb, acc.at[dst_v.at[0]], ss1, add=True)
        pltpu.async_copy(table.at[src_v.at[0]], rows0, gs0)

        @pl.loop(0, NCHS // 2)
        def _main(g):
            for b in range(2):
                ch = g * 2 + b
                rows, gsem, ssem = bufs[b]
                nrows, ngsem, nssem = bufs[1 - b]
                pltpu.make_async_copy(nrows, acc.at[dst_v.at[ch]], nssem).wait()
                nxt = jnp.where(ch + 1 >= NCHS, 0, ch + 1)
                pltpu.async_copy(table.at[src_v.at[nxt]], nrows, ngsem)
                pltpu.make_async_copy(table.at[src_v.at[ch]], rows, gsem).wait()

                base = ch * CH

                @pl.loop(0, CH // L)
                def _grp(jg):
                    nv16 = nrm_v[pl.ds(base + jg * L, L)]
                    r0 = jg * L
                    for q in range(4):
                        nvs = [nv16[jnp.full((L,), 4 * q + t, jnp.int32)]
                               for t in range(4)]
                        for cb in range(HD // L):
                            for t in range(4):
                                slc = pl.ds(cb * L, L)
                                r = r0 + 4 * q + t
                                rows[r, slc] = rows[r, slc] * nvs[t]

                pltpu.async_copy(rows, acc.at[dst_v.at[ch]], ssem, add=True)

        pltpu.make_async_copy(table.at[src_v.at[0]], rows0, gs0).wait()
        pltpu.make_async_copy(rows1, acc.at[dst_v.at[0]], ss1).wait()
        plsc.subcore_barrier()

        pltpu.sync_copy(acc.at[pl.ds(row0, RPT)],
                        out_hbm.at[pl.ds(k * 2 * N_PAD + c * N_PAD + row0, RPT)])
        if k < K - 1:
            for q in range(RPT // CH):
                pltpu.sync_copy(zb, acc.at[pl.ds(row0 + q * CH, CH)])
        plsc.subcore_barrier()


def _sc_layer(hcat, srco3, dst3, nrmf):
    out = pl.kernel(
        _layer_body,
        out_type=jax.ShapeDtypeStruct((K * 2 * N_PAD, HD), jnp.float32),
        mesh=_mesh(),
        compiler_params=pltpu.CompilerParams(
            needs_layout_passes=False, use_tc_tiling_on_sc=False
        ),
        scratch_types=[
            pltpu.VMEM((NCHS, CH), jnp.int32),
            pltpu.VMEM((NCHS, CH), jnp.int32),
            pltpu.VMEM((EPS,), jnp.float32),
            pltpu.VMEM((CH, HD), jnp.float32),
            pltpu.VMEM((CH, HD), jnp.float32),
            pltpu.VMEM((CH, HD), jnp.float32),
            pltpu.VMEM_SHARED((N_PAD, HD), jnp.float32),
            pltpu.SemaphoreType.DMA,
            pltpu.SemaphoreType.DMA,
            pltpu.SemaphoreType.DMA,
            pltpu.SemaphoreType.DMA,
        ],
    )(hcat, srco3, dst3, nrmf)
    return out.reshape(K, 2 * N_PAD, HD)


def _combine_body(*refs):
    hs, wc_ref, b_ref, ost, ofl = refs[: 2 * (K + 1)], refs[-4], refs[-3], refs[-2], refs[-1]
    hblk = jnp.concatenate([h[...].reshape(BN, HD) for h in hs], axis=1)
    acc = jnp.dot(hblk, wc_ref[...], preferred_element_type=jnp.float32)
    acc = acc + b_ref[...]
    acc = jnp.where(acc >= 0, acc, 0.01 * acc)
    ofl[...] = acc
    ost[0] = acc[:, :HD]
    ost[1] = acc[:, HD:]


def _combine(hcat, houts, W, b):
    wc = W.reshape((K + 1) * D, D)
    in_specs = [
        pl.BlockSpec((1, BN, HD), lambda i: (0, i, 0)),
        pl.BlockSpec((1, BN, HD), lambda i: (0, N_PAD // BN + i, 0)),
    ]
    for k in range(K):
        in_specs.append(pl.BlockSpec((1, BN, HD), lambda i, k=k: (k, i, 0)))
        in_specs.append(
            pl.BlockSpec((1, BN, HD), lambda i, k=k: (k, N_PAD // BN + i, 0))
        )
    in_specs.append(pl.BlockSpec(((K + 1) * D, D), lambda i: (0, 0)))
    in_specs.append(pl.BlockSpec((1, D), lambda i: (0, 0)))
    hcat3 = hcat.reshape(1, 2 * N_PAD, HD)
    out_st, out_fl = pl.pallas_call(
        _combine_body,
        grid=(N_PAD // BN,),
        in_specs=in_specs,
        out_specs=[
            pl.BlockSpec((2, BN, HD), lambda i: (0, i, 0)),
            pl.BlockSpec((BN, D), lambda i: (i, 0)),
        ],
        out_shape=[
            jax.ShapeDtypeStruct((2, N_PAD, HD), jnp.float32),
            jax.ShapeDtypeStruct((N_PAD, D), jnp.float32),
        ],
    )(hcat3, hcat3, *[houts for _ in range(2 * K)], wc, b.reshape(1, D))
    return out_st.reshape(2 * N_PAD, HD), out_fl


def kernel(x, edge_index, edge_weight, W1, b1, W2, b2):
    src = edge_index[0]
    dst = edge_index[1]
    norm = _sc_norm(src, dst, edge_weight)

    pad = E_PAD - E
    srcp = jnp.pad(src, (0, pad))
    core_off = (jnp.arange(NC, dtype=jnp.int32) * N_PAD)[:, None]
    srco3 = (srcp[None, :] + core_off).reshape(NC * NS, NCHS, CH)
    dst3 = jnp.pad(dst, (0, pad)).reshape(NS, NCHS, CH)
    nrmf = jnp.pad(norm, (0, pad)).reshape(NS, EPS)

    xp = jnp.pad(x, ((0, N_PAD - N), (0, 0)))
    hcat = jnp.concatenate([xp[:, :HD], xp[:, HD:]], axis=0)

    flat = None
    for (W, b) in ((W1, b1), (W2, b2)):
        houts = _sc_layer(hcat, srco3, dst3, nrmf)
        hcat, flat = _combine(hcat, houts, W, b)
    return flat[:N]
